# baseline (speedup 1.0000x reference)
import jax
import jax.numpy as jnp
from jax.experimental import pallas as pl
from jax.experimental.pallas import tpu as pltpu

_B, _C, _H, _W = 8, 3, 384, 384
_D, _K, _P = 256, 1024, 16
_HP, _WP = _H // _P, _W // _P
_M = _B * _HP * _WP
_NP = _HP * _WP
_CPP = _C * _P * _P


def _vq_body(x_ref, wr_ref, b_ref, cb_ref, out_ref, cbt_ref, cn_ref):
    s = pl.program_id(0)

    @pl.when(s == 0)
    def _prep():
        cbt = cb_ref[...].T
        cbt_ref[...] = cbt
        cn_ref[...] = jnp.sum(cbt * cbt, axis=0, keepdims=True)

    xb = x_ref[0]
    pat = xb.transpose(1, 3, 0, 2, 4).reshape(_NP, _CPP)
    zf = jnp.dot(pat, wr_ref[...], preferred_element_type=jnp.float32)
    zf = zf + b_ref[...]
    d = jnp.dot(zf, cbt_ref[...], preferred_element_type=jnp.float32)
    d = cn_ref[...] - 2.0 * d
    m = jnp.min(d, axis=1, keepdims=True)
    iota = jax.lax.broadcasted_iota(jnp.int32, d.shape, 1)
    idx = jnp.min(jnp.where(d == m, iota, _K), axis=1, keepdims=True)
    out_ref[...] = idx


def kernel(x, W_patch, b_patch, codebook):
    wr = W_patch.reshape(_D, _CPP).T
    x6 = x.reshape(_B, _C, _HP, _P, _WP, _P)
    idx = pl.pallas_call(
        _vq_body,
        grid=(_B,),
        in_specs=[
            pl.BlockSpec((1, _C, _HP, _P, _WP, _P),
                         lambda b: (b, 0, 0, 0, 0, 0)),
            pl.BlockSpec((_CPP, _D), lambda b: (0, 0)),
            pl.BlockSpec((1, _D), lambda b: (0, 0)),
            pl.BlockSpec((_K, _D), lambda b: (0, 0)),
        ],
        out_specs=pl.BlockSpec((_NP, 1), lambda b: (b, 0)),
        out_shape=jax.ShapeDtypeStruct((_M, 1), jnp.int32),
        scratch_shapes=[
            pltpu.VMEM((_D, _K), jnp.float32),
            pltpu.VMEM((1, _K), jnp.float32),
        ],
        compiler_params=pltpu.CompilerParams(
            dimension_semantics=("arbitrary",)),
    )(x6, wr, b_patch.reshape(1, _D), codebook)
    indice = idx.reshape(_B, _NP)
    loss = jnp.array(0.0, dtype=jnp.float32)
    return (loss, indice)


# speedup vs baseline: 2.0216x; 2.0216x over previous
import jax
import jax.numpy as jnp
from jax.experimental import pallas as pl
from jax.experimental.pallas import tpu as pltpu

_B, _C, _H, _W = 8, 3, 384, 384
_D, _K, _P = 256, 1024, 16
_HP, _WP = _H // _P, _W // _P
_M = _B * _HP * _WP
_NP = _HP * _WP
_CPP = _C * _P * _P


def _vq_body(x_ref, wr_ref, b_ref, cb_ref, out_ref, cbt_ref, cn_ref):
    s = pl.program_id(0)

    @pl.when(s == 0)
    def _prep():
        cbt = cb_ref[...].T
        cbt_ref[...] = cbt
        cn_ref[...] = jnp.sum(cbt * cbt, axis=0, keepdims=True)

    xb = x_ref[0]
    y = jnp.swapaxes(xb, 2, 3)
    y5 = y.reshape(_C, _HP, _WP, _P, _P)
    pat = y5.transpose(1, 2, 0, 3, 4).reshape(_NP, _CPP)
    zf = jnp.dot(pat, wr_ref[...], preferred_element_type=jnp.float32)
    zf = zf + b_ref[...]
    d = jnp.dot(zf, cbt_ref[...], preferred_element_type=jnp.float32)
    d = cn_ref[...] - 2.0 * d
    m = jnp.min(d, axis=1, keepdims=True)
    iota = jax.lax.broadcasted_iota(jnp.int32, d.shape, 1)
    idx = jnp.min(jnp.where(d == m, iota, _K), axis=1, keepdims=True)
    out_ref[...] = idx


def kernel(x, W_patch, b_patch, codebook):
    wr = W_patch.transpose(1, 3, 2, 0).reshape(_CPP, _D)
    x6 = x.reshape(_B, _C, _HP, _P, _W)
    idx = pl.pallas_call(
        _vq_body,
        grid=(_B,),
        in_specs=[
            pl.BlockSpec((1, _C, _HP, _P, _W),
                         lambda b: (b, 0, 0, 0, 0)),
            pl.BlockSpec((_CPP, _D), lambda b: (0, 0)),
            pl.BlockSpec((1, _D), lambda b: (0, 0)),
            pl.BlockSpec((_K, _D), lambda b: (0, 0)),
        ],
        out_specs=pl.BlockSpec((_NP, 1), lambda b: (b, 0)),
        out_shape=jax.ShapeDtypeStruct((_M, 1), jnp.int32),
        scratch_shapes=[
            pltpu.VMEM((_D, _K), jnp.float32),
            pltpu.VMEM((1, _K), jnp.float32),
        ],
        compiler_params=pltpu.CompilerParams(
            dimension_semantics=("arbitrary",)),
    )(x6, wr, b_patch.reshape(1, _D), codebook)
    indice = idx.reshape(_B, _NP)
    loss = jnp.array(0.0, dtype=jnp.float32)
    return (loss, indice)
